# baseline (speedup 1.0000x reference)
import functools

import jax
import jax.numpy as jnp
from jax import lax
from jax.experimental import pallas as pl
from jax.experimental.pallas import tpu as pltpu
from jax.experimental.pallas import tpu_sc as plsc

_Lc, _Lf, _N = 4, 8, 32768
_D = 128


def _mlp_body(x_ref, w1t_ref, g1_ref, b1_ref, w2t_ref, g2_ref, b2_ref, o_ref):
    x = x_ref[...]
    hc = jnp.dot(
        x.astype(jnp.bfloat16), w1t_ref[...], preferred_element_type=jnp.float32
    )
    var = jnp.mean(hc * hc, axis=-1, keepdims=True)
    y = hc * lax.rsqrt(var + 1e-5) * g1_ref[...] + b1_ref[...]
    hc2 = jnp.dot(
        y.astype(jnp.bfloat16), w2t_ref[...], preferred_element_type=jnp.float32
    )
    var2 = jnp.mean(hc2 * hc2, axis=-1, keepdims=True)
    o_ref[...] = jnp.maximum(
        hc2 * lax.rsqrt(var2 + 1e-5) * g2_ref[...] + b2_ref[...], 0.0
    )


def _mlp(x, w1, g1, b1, w2, g2, b2, block_rows, out_rows=None, row_offset=0):
    rows, d_in = x.shape
    d_out = w1.shape[0]
    if out_rows is None:
        out_rows = rows
    off_blocks = row_offset // block_rows
    grid = out_rows // block_rows
    w1t = w1.T
    w1t = (w1t - jnp.mean(w1t, axis=1, keepdims=True)).astype(jnp.bfloat16)
    w2t = w2.T
    w2t = (w2t - jnp.mean(w2t, axis=1, keepdims=True)).astype(jnp.bfloat16)
    rep = lambda shape: pl.BlockSpec(shape, lambda i: (0,) * len(shape))
    return pl.pallas_call(
        _mlp_body,
        grid=(grid,),
        in_specs=[
            pl.BlockSpec((block_rows, d_in), lambda i: (i + off_blocks, 0)),
            rep((d_in, d_out)),
            rep((d_out,)),
            rep((d_out,)),
            rep((d_out, d_out)),
            rep((d_out,)),
            rep((d_out,)),
        ],
        out_specs=pl.BlockSpec((block_rows, d_out), lambda i: (i, 0)),
        out_shape=jax.ShapeDtypeStruct((out_rows, d_out), jnp.float32),
        compiler_params=pltpu.CompilerParams(
            dimension_semantics=("arbitrary",),
        ),
    )(x, w1t, g1, b1, w2t, g2, b2)


_NC, _NS, _LANES = 2, 16, 16
_NW = _NC * _NS
_B = _Lc * _N
_CH = 128
_PER_W = _B // _NW
_CHUNKS = _PER_W // _CH


_NBUF = 3


def _scatter_body(n_idx, idx_off, fc_hbm, idx_hbm, out_hbm, *refs):
    per_w = n_idx // _NW
    chunks = per_w // _CH
    wid = lax.axis_index("s") * _NC + lax.axis_index("c")
    base_w = wid * per_w
    idx_v = refs[0:_NBUF]
    fcv = refs[_NBUF:2 * _NBUF]
    rows_v = refs[2 * _NBUF:3 * _NBUF]
    sems = refs[3 * _NBUF:]
    sem_i = sems[0:_NBUF]
    sem_f = sems[_NBUF:2 * _NBUF]
    sem_g = sems[2 * _NBUF:3 * _NBUF]
    sem_s = sems[3 * _NBUF:4 * _NBUF]

    def issue_front(c):
        b = c % _NBUF
        base = base_w + c * _CH
        pltpu.async_copy(
            idx_hbm.at[pl.ds(idx_off + base, _CH)], idx_v[b], sem_i[b]
        ).wait()
        h_fc = pltpu.async_copy(fc_hbm.at[pl.ds(base, _CH)], fcv[b], sem_f[b])
        h_g = pltpu.async_copy(out_hbm.at[idx_v[b]], rows_v[b], sem_g[b])
        return h_fc, h_g

    pending = {c: issue_front(c) for c in range(min(_NBUF - 1, chunks))}
    scatters = [None] * _NBUF
    for c in range(chunks):
        b = c % _NBUF
        nc = c + _NBUF - 1
        if nc < chunks:
            nb = nc % _NBUF
            if scatters[nb] is not None:
                scatters[nb].wait()
                scatters[nb] = None
            pending[nc] = issue_front(nc)
        h_fc, h_g = pending.pop(c)
        h_g.wait()
        h_fc.wait()

        @pl.loop(0, _CH)
        def _row(r):
            for j in range(_D // _LANES):
                sl = pl.ds(j * _LANES, _LANES)
                rows_v[b][r, sl] = rows_v[b][r, sl] + fcv[b][r, sl]

        scatters[b] = pltpu.async_copy(rows_v[b], out_hbm.at[idx_v[b]], sem_s[b])
    for s in scatters:
        if s is not None:
            s.wait()


@functools.cache
def _sc_scatter(n_idx, idx_off):
    return pl.kernel(
        functools.partial(_scatter_body, n_idx, idx_off),
        out_type=(),
        mesh=plsc.VectorSubcoreMesh(
            core_axis_name="c", subcore_axis_name="s",
            num_cores=_NC, num_subcores=_NS,
        ),
        scratch_types=(
            [pltpu.VMEM((_CH,), jnp.int32)] * _NBUF
            + [pltpu.VMEM((_CH, _D), jnp.float32)] * (2 * _NBUF)
            + [pltpu.SemaphoreType.DMA] * (4 * _NBUF)
        ),
    )


def kernel(features_coarse, features_fine, neighbor_idx_0, neighbor_idx_1,
           keep_idx, Wc1, gc1, bc1, Wc2, gc2, bc2, Wf1, gf1, bf1, Wf2, gf2,
           bf2):
    del keep_idx
    ff = _mlp(features_fine.reshape(_Lf * _N, _D), Wf1, gf1, bf1, Wf2, gf2,
              bf2, block_rows=16384)
    flat_idx = neighbor_idx_0 * _N + neighbor_idx_1
    out_ref = jax.new_ref(ff)

    xc = features_coarse.reshape(_Lc * _N, features_coarse.shape[-1])
    part_sizes = (16384, 49152, 49152, 16384)
    off = 0
    for part_rows in part_sizes:
        fc_k = _mlp(xc, Wc1, gc1, bc1, Wc2, gc2, bc2, block_rows=4096,
                    out_rows=part_rows, row_offset=off)
        _sc_scatter(part_rows, off)(fc_k, flat_idx, out_ref)
        off += part_rows
    return out_ref[...].reshape(_Lf, _N, _D)


# speedup vs baseline: 1.0034x; 1.0034x over previous
import functools

import jax
import jax.numpy as jnp
from jax import lax
from jax.experimental import pallas as pl
from jax.experimental.pallas import tpu as pltpu
from jax.experimental.pallas import tpu_sc as plsc

_Lc, _Lf, _N = 4, 8, 32768
_D = 128


def _mlp_body(x_ref, w1t_ref, g1_ref, b1_ref, w2t_ref, g2_ref, b2_ref, o_ref):
    x = x_ref[...]
    hc = jnp.dot(
        x.astype(jnp.bfloat16), w1t_ref[...], preferred_element_type=jnp.float32
    )
    var = jnp.mean(hc * hc, axis=-1, keepdims=True)
    y = hc * lax.rsqrt(var + 1e-5) * g1_ref[...] + b1_ref[...]
    hc2 = jnp.dot(
        y.astype(jnp.bfloat16), w2t_ref[...], preferred_element_type=jnp.float32
    )
    var2 = jnp.mean(hc2 * hc2, axis=-1, keepdims=True)
    o_ref[...] = jnp.maximum(
        hc2 * lax.rsqrt(var2 + 1e-5) * g2_ref[...] + b2_ref[...], 0.0
    ).astype(o_ref.dtype)


def _mlp(x, w1, g1, b1, w2, g2, b2, block_rows, out_rows=None, row_offset=0,
         out_dtype=jnp.float32, interleave_out=False):
    rows, d_in = x.shape
    d_out = w1.shape[0]
    if out_rows is None:
        out_rows = rows
    off_blocks = row_offset // block_rows
    grid = out_rows // block_rows
    w1t = w1.T
    w1t = (w1t - jnp.mean(w1t, axis=1, keepdims=True)).astype(jnp.bfloat16)
    w2t = w2.T
    w2t = (w2t - jnp.mean(w2t, axis=1, keepdims=True)).astype(jnp.bfloat16)
    if interleave_out:
        perm = jnp.arange(d_out).reshape(d_out // 32, 2, 16)
        perm = jnp.moveaxis(perm, -1, -2).reshape(d_out)
        w2t = w2t[:, perm]
        g2 = g2[perm]
        b2 = b2[perm]
    rep = lambda shape: pl.BlockSpec(shape, lambda i: (0,) * len(shape))
    return pl.pallas_call(
        _mlp_body,
        grid=(grid,),
        in_specs=[
            pl.BlockSpec((block_rows, d_in), lambda i: (i + off_blocks, 0)),
            rep((d_in, d_out)),
            rep((d_out,)),
            rep((d_out,)),
            rep((d_out, d_out)),
            rep((d_out,)),
            rep((d_out,)),
        ],
        out_specs=pl.BlockSpec((block_rows, d_out), lambda i: (i, 0)),
        out_shape=jax.ShapeDtypeStruct((out_rows, d_out), out_dtype),
        compiler_params=pltpu.CompilerParams(
            dimension_semantics=("arbitrary",),
        ),
    )(x, w1t, g1, b1, w2t, g2, b2)


_NC, _NS, _LANES = 2, 16, 16
_NW = _NC * _NS
_B = _Lc * _N
_CH = 128
_PER_W = _B // _NW
_CHUNKS = _PER_W // _CH


_NBUF = 3


def _scatter_body(n_idx, idx_off, fc_hbm, idx_hbm, out_hbm, *refs):
    per_w = n_idx // _NW
    chunks = per_w // _CH
    wid = lax.axis_index("s") * _NC + lax.axis_index("c")
    base_w = wid * per_w
    idx_v = refs[0:_NBUF]
    fcv = refs[_NBUF:2 * _NBUF]
    rows_v = refs[2 * _NBUF:3 * _NBUF]
    sems = refs[3 * _NBUF:]
    sem_i = sems[0:_NBUF]
    sem_f = sems[_NBUF:2 * _NBUF]
    sem_g = sems[2 * _NBUF:3 * _NBUF]
    sem_s = sems[3 * _NBUF:4 * _NBUF]

    def issue_front(c):
        b = c % _NBUF
        base = base_w + c * _CH
        pltpu.async_copy(
            idx_hbm.at[pl.ds(idx_off + base, _CH)], idx_v[b], sem_i[b]
        ).wait()
        h_fc = pltpu.async_copy(fc_hbm.at[pl.ds(base, _CH)], fcv[b], sem_f[b])
        h_g = pltpu.async_copy(out_hbm.at[idx_v[b]], rows_v[b], sem_g[b])
        return h_fc, h_g

    pending = {c: issue_front(c) for c in range(min(_NBUF - 1, chunks))}
    scatters = [None] * _NBUF
    for c in range(chunks):
        b = c % _NBUF
        nc = c + _NBUF - 1
        if nc < chunks:
            nb = nc % _NBUF
            if scatters[nb] is not None:
                scatters[nb].wait()
                scatters[nb] = None
            pending[nc] = issue_front(nc)
        h_fc, h_g = pending.pop(c)
        h_g.wait()
        h_fc.wait()

        @pl.loop(0, _CH)
        def _row(r):
            for j in range(_D // _LANES):
                sl = pl.ds(j * _LANES, _LANES)
                rows_v[b][r, sl] = rows_v[b][r, sl] + fcv[b][r, sl]

        scatters[b] = pltpu.async_copy(rows_v[b], out_hbm.at[idx_v[b]], sem_s[b])
    for s in scatters:
        if s is not None:
            s.wait()


@functools.cache
def _sc_scatter(n_idx, idx_off):
    return pl.kernel(
        functools.partial(_scatter_body, n_idx, idx_off),
        out_type=(),
        mesh=plsc.VectorSubcoreMesh(
            core_axis_name="c", subcore_axis_name="s",
            num_cores=_NC, num_subcores=_NS,
        ),
        scratch_types=(
            [pltpu.VMEM((_CH,), jnp.int32)] * _NBUF
            + [pltpu.VMEM((_CH, _D), jnp.float32)] * _NBUF
            + [pltpu.VMEM((_CH, _D), jnp.float32)] * _NBUF
            + [pltpu.SemaphoreType.DMA] * (4 * _NBUF)
        ),
    )


def kernel(features_coarse, features_fine, neighbor_idx_0, neighbor_idx_1,
           keep_idx, Wc1, gc1, bc1, Wc2, gc2, bc2, Wf1, gf1, bf1, Wf2, gf2,
           bf2):
    del keep_idx
    ff = _mlp(features_fine.reshape(_Lf * _N, _D), Wf1, gf1, bf1, Wf2, gf2,
              bf2, block_rows=16384)
    flat_idx = neighbor_idx_0 * _N + neighbor_idx_1
    out_ref = jax.new_ref(ff)

    xc = features_coarse.reshape(_Lc * _N, features_coarse.shape[-1])
    part_sizes = (16384, 49152, 49152, 16384)
    off = 0
    for part_rows in part_sizes:
        fc_k = _mlp(xc, Wc1, gc1, bc1, Wc2, gc2, bc2, block_rows=4096,
                    out_rows=part_rows, row_offset=off,
                    )
        _sc_scatter(part_rows, off)(fc_k, flat_idx, out_ref)
        off += part_rows
    return out_ref[...].reshape(_Lf, _N, _D)
